# baseline (speedup 1.0000x reference)
import functools

import jax
import jax.numpy as jnp
from jax import lax
from jax.experimental import pallas as pl
from jax.experimental.pallas import tpu as pltpu
from jax.experimental.pallas import tpu_sc as plsc

B = 16384
D = 64
NC, NS = 2, 16
NW = NC * NS
BPW = B // NW
BS = 1024


def _gather_body(uidx_hbm, midx_hbm, utab_hbm, itab_hbm, uout_hbm, vout_hbm,
                 uidx_v, midx_v, urows_v, vrows_v, sem_u, sem_v):
    wid = lax.axis_index("s") * NC + lax.axis_index("c")
    base = wid * BPW
    pltpu.sync_copy(uidx_hbm.at[pl.ds(base, BPW)], uidx_v)
    pltpu.sync_copy(midx_hbm.at[pl.ds(base, BPW)], midx_v)
    cu = pltpu.async_copy(utab_hbm.at[uidx_v], urows_v, sem_u)
    cv = pltpu.async_copy(itab_hbm.at[midx_v], vrows_v, sem_v)
    cu.wait()
    cv.wait()
    pltpu.sync_copy(urows_v, uout_hbm.at[pl.ds(base, BPW)])
    pltpu.sync_copy(vrows_v, vout_hbm.at[pl.ds(base, BPW)])


def _sc_gather(user_idx, movie_idx, user_emb, item_emb):
    mesh = plsc.VectorSubcoreMesh(core_axis_name="c", subcore_axis_name="s")
    f = pl.kernel(
        _gather_body,
        mesh=mesh,
        compiler_params=pltpu.CompilerParams(use_tc_tiling_on_sc=False),
        out_type=[
            jax.ShapeDtypeStruct((B, D), jnp.float32),
            jax.ShapeDtypeStruct((B, D), jnp.float32),
        ],
        scratch_types=[
            pltpu.VMEM((BPW,), jnp.int32),
            pltpu.VMEM((BPW,), jnp.int32),
            pltpu.VMEM((BPW, D), jnp.float32),
            pltpu.VMEM((BPW, D), jnp.float32),
            pltpu.SemaphoreType.DMA,
            pltpu.SemaphoreType.DMA,
        ],
    )
    return f(user_idx, movie_idx, user_emb, item_emb)


def _mlp_body(u_ref, v_ref, w1_ref, b1_ref, w2_ref, b2_ref, w3_ref, b3_ref,
              o_ref):
    u = u_ref[...]
    v = v_ref[...]
    x = jnp.concatenate([u, v, u * v, jnp.abs(u - v)], axis=1)
    h = jnp.dot(x, w1_ref[...], preferred_element_type=jnp.float32) + b1_ref[...]
    h = jnp.maximum(h, 0.0)
    h = jnp.dot(h, w2_ref[...], preferred_element_type=jnp.float32) + b2_ref[...]
    h = jnp.maximum(h, 0.0)
    o_ref[...] = jnp.sum(h * w3_ref[...], axis=1) + b3_ref[0]


def _tc_mlp(u_g, v_g, w1t, b1, w2t, b2, w3, b3):
    grid = (B // BS,)
    return pl.pallas_call(
        _mlp_body,
        grid=grid,
        in_specs=[
            pl.BlockSpec((BS, D), lambda i: (i, 0)),
            pl.BlockSpec((BS, D), lambda i: (i, 0)),
            pl.BlockSpec((256, 256), lambda i: (0, 0)),
            pl.BlockSpec((1, 256), lambda i: (0, 0)),
            pl.BlockSpec((256, 64), lambda i: (0, 0)),
            pl.BlockSpec((1, 64), lambda i: (0, 0)),
            pl.BlockSpec((1, 64), lambda i: (0, 0)),
            pl.BlockSpec(memory_space=pltpu.SMEM),
        ],
        out_specs=pl.BlockSpec((BS,), lambda i: (i,)),
        out_shape=jax.ShapeDtypeStruct((B,), jnp.float32),
    )(u_g, v_g, w1t, b1, w2t, b2, w3, b3)


def kernel(user_idx, movie_idx, user_emb, item_emb, W1, b1, W2, b2, W3, b3):
    u_g, v_g = _sc_gather(user_idx, movie_idx, user_emb, item_emb)
    return _tc_mlp(u_g, v_g, W1.T, b1.reshape(1, 256), W2.T, b2.reshape(1, 64),
                   W3, b3)


# speedup vs baseline: 1.5676x; 1.5676x over previous
import functools

import jax
import jax.numpy as jnp
from jax import lax
from jax.experimental import pallas as pl
from jax.experimental.pallas import tpu as pltpu
from jax.experimental.pallas import tpu_sc as plsc

B = 16384
D = 64
NC, NS = 2, 16
NW = NC * NS
BPW = B // NW
BS = 1024
CH = 256


def _gather_body(uidx_hbm, midx_hbm, utab_hbm, itab_hbm, uout_hbm, vout_hbm,
                 uidx_v, midx_v, urows_v, vrows_v, sem_u, sem_v):
    wid = lax.axis_index("s") * NC + lax.axis_index("c")
    base = wid * BPW
    pltpu.sync_copy(uidx_hbm.at[pl.ds(base, BPW)], uidx_v)
    pltpu.sync_copy(midx_hbm.at[pl.ds(base, BPW)], midx_v)
    lanes = lax.broadcasted_iota(jnp.int32, (16,), 0)

    for c in range(BPW // CH):
        off = c * CH

        def fire(k, _):
            vec_u = uidx_v[pl.ds(off + k * 16, 16)]
            vec_v = midx_v[pl.ds(off + k * 16, 16)]
            for l in range(16):
                iu = jnp.sum(jnp.where(lanes == l, vec_u, 0))
                iv = jnp.sum(jnp.where(lanes == l, vec_v, 0))
                pltpu.async_copy(utab_hbm.at[pl.ds(iu, 1), :],
                                 urows_v.at[pl.ds(k * 16 + l, 1), :], sem_u)
                pltpu.async_copy(itab_hbm.at[pl.ds(iv, 1), :],
                                 vrows_v.at[pl.ds(k * 16 + l, 1), :], sem_v)
            return 0

        lax.fori_loop(0, CH // 16, fire, 0)

        def drain(j, _):
            pltpu.make_async_copy(utab_hbm.at[pl.ds(0, 1), :],
                                  urows_v.at[pl.ds(j, 1), :], sem_u).wait()
            pltpu.make_async_copy(itab_hbm.at[pl.ds(0, 1), :],
                                  vrows_v.at[pl.ds(j, 1), :], sem_v).wait()
            return 0

        lax.fori_loop(0, CH, drain, 0)
        pltpu.sync_copy(urows_v, uout_hbm.at[pl.ds(base + off, CH)])
        pltpu.sync_copy(vrows_v, vout_hbm.at[pl.ds(base + off, CH)])


def _sc_gather(user_idx, movie_idx, user_emb, item_emb):
    mesh = plsc.VectorSubcoreMesh(core_axis_name="c", subcore_axis_name="s")
    f = pl.kernel(
        _gather_body,
        mesh=mesh,
        compiler_params=pltpu.CompilerParams(needs_layout_passes=False),
        out_type=[
            jax.ShapeDtypeStruct((B, D), jnp.float32),
            jax.ShapeDtypeStruct((B, D), jnp.float32),
        ],
        scratch_types=[
            pltpu.VMEM((BPW,), jnp.int32),
            pltpu.VMEM((BPW,), jnp.int32),
            pltpu.VMEM((CH, D), jnp.float32),
            pltpu.VMEM((CH, D), jnp.float32),
            pltpu.SemaphoreType.DMA,
            pltpu.SemaphoreType.DMA,
        ],
    )
    return f(user_idx, movie_idx, user_emb, item_emb)


def _mlp_body(u_ref, v_ref, w1_ref, b1_ref, w2_ref, b2_ref, w3_ref, b3_ref,
              o_ref):
    u = u_ref[...]
    v = v_ref[...]
    x = jnp.concatenate([u, v, u * v, jnp.abs(u - v)], axis=1)
    h = jnp.dot(x, w1_ref[...], preferred_element_type=jnp.float32) + b1_ref[...]
    h = jnp.maximum(h, 0.0)
    h = jnp.dot(h, w2_ref[...], preferred_element_type=jnp.float32) + b2_ref[...]
    h = jnp.maximum(h, 0.0)
    o_ref[...] = jnp.sum(h * w3_ref[...], axis=1) + b3_ref[0]


def _tc_mlp(u_g, v_g, w1t, b1, w2t, b2, w3, b3):
    grid = (B // BS,)
    return pl.pallas_call(
        _mlp_body,
        grid=grid,
        in_specs=[
            pl.BlockSpec((BS, D), lambda i: (i, 0)),
            pl.BlockSpec((BS, D), lambda i: (i, 0)),
            pl.BlockSpec((256, 256), lambda i: (0, 0)),
            pl.BlockSpec((1, 256), lambda i: (0, 0)),
            pl.BlockSpec((256, 64), lambda i: (0, 0)),
            pl.BlockSpec((1, 64), lambda i: (0, 0)),
            pl.BlockSpec((1, 64), lambda i: (0, 0)),
            pl.BlockSpec(memory_space=pltpu.SMEM),
        ],
        out_specs=pl.BlockSpec((BS,), lambda i: (i,)),
        out_shape=jax.ShapeDtypeStruct((B,), jnp.float32),
    )(u_g, v_g, w1t, b1, w2t, b2, w3, b3)


def kernel(user_idx, movie_idx, user_emb, item_emb, W1, b1, W2, b2, W3, b3):
    u_g, v_g = _sc_gather(user_idx, movie_idx, user_emb, item_emb)
    return _tc_mlp(u_g, v_g, W1.T, b1.reshape(1, 256), W2.T, b2.reshape(1, 64),
                   W3, b3)


# speedup vs baseline: 1.5685x; 1.0006x over previous
import functools

import jax
import jax.numpy as jnp
from jax import lax
from jax.experimental import pallas as pl
from jax.experimental.pallas import tpu as pltpu
from jax.experimental.pallas import tpu_sc as plsc

B = 16384
D = 64
NC, NS = 2, 16
NW = NC * NS
BPW = B // NW
BS = 1024
CH = 256


def _gather_body(uidx_hbm, midx_hbm, utab_hbm, itab_hbm, uout_hbm, vout_hbm,
                 uidx_v, midx_v, urows_v, vrows_v, sem_u, sem_v):
    wid = lax.axis_index("s") * NC + lax.axis_index("c")
    base = wid * BPW
    pltpu.sync_copy(uidx_hbm.at[pl.ds(base, BPW)], uidx_v)
    pltpu.sync_copy(midx_hbm.at[pl.ds(base, BPW)], midx_v)
    lanes = lax.broadcasted_iota(jnp.int32, (16,), 0)

    for c in range(BPW // CH):
        off = c * CH

        def fire(k, _):
            vec_u = uidx_v[pl.ds(off + k * 16, 16)]
            vec_v = midx_v[pl.ds(off + k * 16, 16)]
            for l in range(16):
                iu = jnp.sum(jnp.where(lanes == l, vec_u, 0))
                iv = jnp.sum(jnp.where(lanes == l, vec_v, 0))
                pltpu.async_copy(utab_hbm.at[pl.ds(iu, 1), :],
                                 urows_v.at[pl.ds(k * 16 + l, 1), :], sem_u)
                pltpu.async_copy(itab_hbm.at[pl.ds(iv, 1), :],
                                 vrows_v.at[pl.ds(k * 16 + l, 1), :], sem_v)
            return 0

        lax.fori_loop(0, CH // 16, fire, 0)

        def drain(j, _):
            pltpu.make_async_copy(utab_hbm.at[pl.ds(0, 1), :],
                                  urows_v.at[pl.ds(j, 1), :], sem_u).wait()
            pltpu.make_async_copy(itab_hbm.at[pl.ds(0, 1), :],
                                  vrows_v.at[pl.ds(j, 1), :], sem_v).wait()
            return 0

        lax.fori_loop(0, CH, drain, 0)
        pltpu.sync_copy(urows_v, uout_hbm.at[pl.ds(base + off, CH)])
        pltpu.sync_copy(vrows_v, vout_hbm.at[pl.ds(base + off, CH)])


def _sc_gather(user_idx, movie_idx, user_emb, item_emb):
    mesh = plsc.VectorSubcoreMesh(core_axis_name="c", subcore_axis_name="s")
    f = pl.kernel(
        _gather_body,
        mesh=mesh,
        compiler_params=pltpu.CompilerParams(needs_layout_passes=False, use_tc_tiling_on_sc=True),
        out_type=[
            jax.ShapeDtypeStruct((B, D), jnp.float32),
            jax.ShapeDtypeStruct((B, D), jnp.float32),
        ],
        scratch_types=[
            pltpu.VMEM((BPW,), jnp.int32),
            pltpu.VMEM((BPW,), jnp.int32),
            pltpu.VMEM((CH, D), jnp.float32),
            pltpu.VMEM((CH, D), jnp.float32),
            pltpu.SemaphoreType.DMA,
            pltpu.SemaphoreType.DMA,
        ],
    )
    return f(user_idx, movie_idx, user_emb, item_emb)


def _mlp_body(u_ref, v_ref, w1_ref, b1_ref, w2_ref, b2_ref, w3_ref, b3_ref,
              o_ref):
    u = u_ref[...]
    v = v_ref[...]
    x = jnp.concatenate([u, v, u * v, jnp.abs(u - v)], axis=1)
    h = jnp.dot(x, w1_ref[...], preferred_element_type=jnp.float32) + b1_ref[...]
    h = jnp.maximum(h, 0.0)
    h = jnp.dot(h, w2_ref[...], preferred_element_type=jnp.float32) + b2_ref[...]
    h = jnp.maximum(h, 0.0)
    o_ref[...] = jnp.sum(h * w3_ref[...], axis=1) + b3_ref[0]


def _tc_mlp(u_g, v_g, w1t, b1, w2t, b2, w3, b3):
    grid = (B // BS,)
    return pl.pallas_call(
        _mlp_body,
        grid=grid,
        in_specs=[
            pl.BlockSpec((BS, D), lambda i: (i, 0)),
            pl.BlockSpec((BS, D), lambda i: (i, 0)),
            pl.BlockSpec((256, 256), lambda i: (0, 0)),
            pl.BlockSpec((1, 256), lambda i: (0, 0)),
            pl.BlockSpec((256, 64), lambda i: (0, 0)),
            pl.BlockSpec((1, 64), lambda i: (0, 0)),
            pl.BlockSpec((1, 64), lambda i: (0, 0)),
            pl.BlockSpec(memory_space=pltpu.SMEM),
        ],
        out_specs=pl.BlockSpec((BS,), lambda i: (i,)),
        out_shape=jax.ShapeDtypeStruct((B,), jnp.float32),
    )(u_g, v_g, w1t, b1, w2t, b2, w3, b3)


def kernel(user_idx, movie_idx, user_emb, item_emb, W1, b1, W2, b2, W3, b3):
    u_g, v_g = _sc_gather(user_idx, movie_idx, user_emb, item_emb)
    return _tc_mlp(u_g, v_g, W1.T, b1.reshape(1, 256), W2.T, b2.reshape(1, 64),
                   W3, b3)
